# baseline (speedup 1.0000x reference)
import functools

import jax
import jax.numpy as jnp
from jax import lax
from jax.experimental import pallas as pl
from jax.experimental.pallas import tpu as pltpu
from jax.experimental.pallas import tpu_sc as plsc

BB = 4096
EE = 16
NA = 8192
TM = 128
SLOTS = 10240
NTILES = SLOTS // TM
CAP = 4352
DPAD = 640
DBLK = 256
TB = 256
T1 = BB // TB

_sc_mesh = plsc.VectorSubcoreMesh(core_axis_name="c", subcore_axis_name="s")
_sc_params = pltpu.CompilerParams(needs_layout_passes=False)


def _k1_body(x_ref, Wg_ref, bg_ref, eidx_ref, gates_ref, hist_ref, hacc):
    t = pl.program_id(0)
    x = x_ref[...]
    logits = jnp.dot(x, Wg_ref[...], preferred_element_type=jnp.float32)
    logits = logits + bg_ref[...]
    m = jnp.max(logits, axis=1, keepdims=True)
    p = jnp.exp(logits - m)
    p = p / jnp.sum(p, axis=1, keepdims=True)
    lane = lax.broadcasted_iota(jnp.int32, (TB, EE), 1)
    m1 = jnp.max(p, axis=1, keepdims=True)
    a1 = jnp.min(jnp.where(p == m1, lane, EE), axis=1, keepdims=True)
    p2 = jnp.where(lane == a1, -1.0, p)
    m2 = jnp.max(p2, axis=1, keepdims=True)
    a2 = jnp.min(jnp.where(p2 == m2, lane, EE), axis=1, keepdims=True)
    s = m1 + m2
    eidx_ref[...] = jnp.concatenate([a1, a2], axis=1)
    gates_ref[...] = jnp.concatenate([m1 / s, m2 / s], axis=1)
    oh = (lane == a1).astype(jnp.int32) + (lane == a2).astype(jnp.int32)
    contrib = jnp.sum(oh, axis=0, keepdims=True)

    @pl.when(t == 0)
    def _():
        hacc[...] = contrib

    @pl.when(t > 0)
    def _():
        hacc[...] += contrib

    @pl.when(t == T1 - 1)
    def _():
        hist_ref[...] = hacc[...]


def _k1(x, Wg_p, bg):
    return pl.pallas_call(
        _k1_body,
        grid=(T1,),
        in_specs=[
            pl.BlockSpec((TB, DPAD), lambda t: (t, 0)),
            pl.BlockSpec((DPAD, EE), lambda t: (0, 0)),
            pl.BlockSpec((1, EE), lambda t: (0, 0)),
        ],
        out_specs=[
            pl.BlockSpec((TB, 2), lambda t: (t, 0)),
            pl.BlockSpec((TB, 2), lambda t: (t, 0)),
            pl.BlockSpec((1, EE), lambda t: (0, 0)),
        ],
        out_shape=[
            jax.ShapeDtypeStruct((BB, 2), jnp.int32),
            jax.ShapeDtypeStruct((BB, 2), jnp.float32),
            jax.ShapeDtypeStruct((1, EE), jnp.int32),
        ],
        scratch_shapes=[pltpu.VMEM((1, EE), jnp.int32)],
        compiler_params=pltpu.CompilerParams(
            dimension_semantics=("arbitrary",)),
    )(x, Wg_p, bg.reshape(1, EE))


@functools.partial(
    pl.kernel, mesh=_sc_mesh,
    out_type=[
        jax.ShapeDtypeStruct((SLOTS,), jnp.int32),
        jax.ShapeDtypeStruct((NA + 128,), jnp.int32),
        jax.ShapeDtypeStruct((144,), jnp.int32),
    ],
    scratch_types=[
        pltpu.VMEM((NA,), jnp.int32),
        pltpu.VMEM((16,), jnp.int32),
        pltpu.VMEM((CAP,), jnp.int32),
        pltpu.VMEM((CAP,), jnp.int32),
        pltpu.VMEM((CAP,), jnp.int32),
        pltpu.VMEM((16,), jnp.int32),
        pltpu.SemaphoreType.DMA,
    ],
    compiler_params=_sc_params,
)
def _k2a(eidx_hbm, hist_hbm, tok_hbm, pos_hbm, tile_hbm,
         a_v, hist_v, buf_v, pv_v, tok_v, tv_v, sem):
    w = lax.axis_index("s") * 2 + lax.axis_index("c")

    @pl.when(w < EE)
    def _():
        lane = lax.iota(jnp.int32, 16)
        pltpu.sync_copy(eidx_hbm, a_v)
        pltpu.sync_copy(hist_hbm, hist_v)
        hist = hist_v[...]
        pc = jnp.bitwise_and(hist + (TM - 1), -TM)
        incl = plsc.cumsum(pc)
        off_all = incl - pc
        off_s = jnp.sum(jnp.where(lane == w, off_all, 0))
        ntiles_s = jnp.sum(jnp.where(lane == w, pc, 0)) >> 7

        def init_body(j, _):
            buf_v[pl.ds(j * 16, 16)] = NA + ((j & 7) * 16) + lane
            return 0
        lax.fori_loop(0, CAP // 16, init_body, 0)

        def comp_body(g, cur):
            av = a_v[pl.ds(g * 16, 16)]
            mk = av == w
            mi = mk.astype(jnp.int32)
            r = plsc.cumsum(mi)
            ids = g * 16 + lane
            plsc.store_scatter(buf_v, [cur + r - 1], ids, mask=mk)
            return cur + jnp.sum(mi)
        lax.fori_loop(0, NA // 16, comp_body, jnp.zeros((16,), jnp.int32))

        def pv_body(j, _):
            pv_v[pl.ds(j * 16, 16)] = off_s + j * 16 + lane
            tok_v[pl.ds(j * 16, 16)] = buf_v[pl.ds(j * 16, 16)] >> 1
            return 0
        lax.fori_loop(0, CAP // 16, pv_body, 0)

        pltpu.async_copy(pv_v, pos_hbm.at[buf_v], sem).wait()

        def tk_body(j, _):
            dst = pl.multiple_of(off_s + j * TM, TM)
            pltpu.sync_copy(tok_v.at[pl.ds(j * TM, TM)],
                            tok_hbm.at[pl.ds(dst, TM)])
            return 0
        lax.fori_loop(0, ntiles_s, tk_body, 0)

        tv_v[...] = jnp.broadcast_to(w, (16,)).astype(jnp.int32)
        for j2 in range(2):
            tidx = jnp.where(j2 * 16 + lane < ntiles_s,
                             (off_s >> 7) + j2 * 16 + lane, 128 + lane)
            pltpu.async_copy(tv_v, tile_hbm.at[tidx], sem).wait()


@functools.partial(
    pl.kernel, mesh=_sc_mesh,
    out_type=jax.ShapeDtypeStruct((SLOTS, DPAD), jnp.float32),
    scratch_types=[
        pltpu.VMEM((80,), jnp.int32),
        pltpu.VMEM((80, DPAD), jnp.float32),
        pltpu.SemaphoreType.DMA,
    ],
    compiler_params=_sc_params,
)
def _k2b(tok_hbm, x_hbm, xs_hbm, idx_v, rows_v, sem):
    w = lax.axis_index("s") * 2 + lax.axis_index("c")
    base = w * (SLOTS // 32)
    for cch in range(4):
        pltpu.sync_copy(tok_hbm.at[pl.ds(base + cch * 80, 80)], idx_v)
        for j in range(5):
            idx_v[pl.ds(j * 16, 16)] = idx_v[pl.ds(j * 16, 16)] & (BB - 1)
        pltpu.async_copy(x_hbm.at[idx_v], rows_v, sem).wait()
        pltpu.sync_copy(rows_v, xs_hbm.at[pl.ds(base + cch * 80, 80)])


def _k3_body(tile_ref, xs_ref, W1_ref, b1_ref, W2_ref, b2_ref, Wout_ref,
             bout_ref, z_ref):
    t = pl.program_id(0)
    e = tile_ref[t] & (EE - 1)
    x = xs_ref[...]
    h = jnp.dot(x, W1_ref[e], preferred_element_type=jnp.float32) + b1_ref[e]
    h = jnp.maximum(h, 0.0)
    h = jnp.dot(h, W2_ref[e], preferred_element_type=jnp.float32) + b2_ref[e]
    h = jnp.maximum(h, 0.0)
    z_ref[...] = (jnp.dot(h, Wout_ref[...], preferred_element_type=jnp.float32)
                  + bout_ref[...])


def _k3(tile_e, xs, W1_p, b1, W2, b2, Wout, bout):
    return pl.pallas_call(
        _k3_body,
        grid=(NTILES,),
        in_specs=[
            pl.BlockSpec(memory_space=pltpu.MemorySpace.SMEM),
            pl.BlockSpec((TM, DPAD), lambda t: (t, 0)),
            pl.BlockSpec((EE, DPAD, DBLK), lambda t: (0, 0, 0)),
            pl.BlockSpec((EE, 1, DBLK), lambda t: (0, 0, 0)),
            pl.BlockSpec((EE, DBLK, DBLK), lambda t: (0, 0, 0)),
            pl.BlockSpec((EE, 1, DBLK), lambda t: (0, 0, 0)),
            pl.BlockSpec((DBLK, 1), lambda t: (0, 0)),
            pl.BlockSpec((1, 1), lambda t: (0, 0)),
        ],
        out_specs=pl.BlockSpec((TM, 1), lambda t: (t, 0)),
        out_shape=jax.ShapeDtypeStruct((SLOTS, 1), jnp.float32),
        compiler_params=pltpu.CompilerParams(
            dimension_semantics=("arbitrary",)),
    )(tile_e, xs, W1_p, b1.reshape(EE, 1, DBLK), W2,
      b2.reshape(EE, 1, DBLK), Wout, bout.reshape(1, 1))


@functools.partial(
    pl.kernel, mesh=_sc_mesh,
    out_type=jax.ShapeDtypeStruct((BB,), jnp.float32),
    scratch_types=[
        pltpu.VMEM((SLOTS,), jnp.float32),
        pltpu.VMEM((256,), jnp.int32),
        pltpu.VMEM((256,), jnp.float32),
        pltpu.VMEM((256,), jnp.float32),
        pltpu.VMEM((128,), jnp.float32),
        pltpu.SemaphoreType.DMA,
    ],
    compiler_params=_sc_params,
)
def _k4(z_hbm, pos_hbm, g_hbm, out_hbm, z_v, pos_v, g_v, val_v, out_v, sem):
    w = lax.axis_index("s") * 2 + lax.axis_index("c")
    lane = lax.iota(jnp.int32, 16)
    pltpu.sync_copy(z_hbm, z_v)
    pltpu.sync_copy(pos_hbm.at[pl.ds(w * 256, 256)], pos_v)
    pltpu.sync_copy(g_hbm.at[pl.ds(w * 256, 256)], g_v)
    for j in range(16):
        idx = pos_v[pl.ds(j * 16, 16)]
        val_v[pl.ds(j * 16, 16)] = (
            plsc.load_gather(z_v, [idx]) * g_v[pl.ds(j * 16, 16)])
    for j in range(8):
        idxe = j * 32 + 2 * lane
        oe = plsc.load_gather(val_v, [idxe])
        oo = plsc.load_gather(val_v, [idxe + 1])
        out_v[pl.ds(j * 16, 16)] = oe + oo
    pltpu.sync_copy(out_v, out_hbm.at[pl.ds(w * 128, 128)])


@jax.jit
def _moe_sparse(x_num, x_cat, Wg, bg, W1, b1, W2, b2, Wout, bout):
    B = x_num.shape[0]
    oh = jax.nn.one_hot(x_cat, 16, dtype=jnp.float32)
    oh = oh.reshape(B, x_cat.shape[1] * 16)
    x = jnp.concatenate(
        [x_num, oh, jnp.zeros((B, DPAD - 544), jnp.float32)], axis=1)
    Wg_p = jnp.concatenate(
        [Wg, jnp.zeros((DPAD - 544, EE), jnp.float32)], axis=0)
    W1_p = jnp.concatenate(
        [W1, jnp.zeros((EE, DPAD - 544, DBLK), jnp.float32)], axis=1)

    eidx, gates, hist = _k1(x, Wg_p, bg)
    tok, pos, tile_e = _k2a(eidx.reshape(-1), hist.reshape(-1))
    xs = _k2b(tok, x)
    z = _k3(tile_e, xs, W1_p, b1, W2, b2, Wout, bout)
    out = _k4(z.reshape(-1), pos, gates.reshape(-1))
    return out.reshape(B, 1, 1)


def kernel(x_num, x_cat, Wg, bg, W1, b1, W2, b2, Wout, bout):
    return _moe_sparse(x_num, x_cat, Wg, bg, W1, b1, W2, b2, Wout, bout)


# speedup vs baseline: 39.0418x; 39.0418x over previous
import functools

import jax
import jax.numpy as jnp
from jax import lax
from jax.experimental import pallas as pl
from jax.experimental.pallas import tpu as pltpu
from jax.experimental.pallas import tpu_sc as plsc

BB = 4096
EE = 16
NA = 8192
TM = 128
SLOTS = 10240
NTILES = SLOTS // TM
CAP = 4352
DPAD = 640
DBLK = 256
TB = 256
T1 = BB // TB

_sc_mesh = plsc.VectorSubcoreMesh(core_axis_name="c", subcore_axis_name="s")
_sc_params = pltpu.CompilerParams(needs_layout_passes=False)


def _k1_body(x_ref, Wg_ref, bg_ref, eidx_ref, gates_ref, hist_ref, hacc):
    t = pl.program_id(0)
    x = x_ref[...]
    logits = jnp.dot(x, Wg_ref[...], preferred_element_type=jnp.float32)
    logits = logits + bg_ref[...]
    m = jnp.max(logits, axis=1, keepdims=True)
    p = jnp.exp(logits - m)
    p = p / jnp.sum(p, axis=1, keepdims=True)
    lane = lax.broadcasted_iota(jnp.int32, (TB, EE), 1)
    m1 = jnp.max(p, axis=1, keepdims=True)
    a1 = jnp.min(jnp.where(p == m1, lane, EE), axis=1, keepdims=True)
    p2 = jnp.where(lane == a1, -1.0, p)
    m2 = jnp.max(p2, axis=1, keepdims=True)
    a2 = jnp.min(jnp.where(p2 == m2, lane, EE), axis=1, keepdims=True)
    s = m1 + m2
    eidx_ref[...] = jnp.concatenate([a1, a2], axis=1)
    gates_ref[...] = jnp.concatenate([m1 / s, m2 / s], axis=1)
    oh = (lane == a1).astype(jnp.int32) + (lane == a2).astype(jnp.int32)
    contrib = jnp.sum(oh, axis=0, keepdims=True)

    @pl.when(t == 0)
    def _():
        hacc[...] = contrib

    @pl.when(t > 0)
    def _():
        hacc[...] += contrib

    @pl.when(t == T1 - 1)
    def _():
        hist_ref[...] = hacc[...]


def _k1(x, Wg_p, bg):
    return pl.pallas_call(
        _k1_body,
        grid=(T1,),
        in_specs=[
            pl.BlockSpec((TB, DPAD), lambda t: (t, 0)),
            pl.BlockSpec((DPAD, EE), lambda t: (0, 0)),
            pl.BlockSpec((1, EE), lambda t: (0, 0)),
        ],
        out_specs=[
            pl.BlockSpec((TB, 2), lambda t: (t, 0)),
            pl.BlockSpec((TB, 2), lambda t: (t, 0)),
            pl.BlockSpec((1, EE), lambda t: (0, 0)),
        ],
        out_shape=[
            jax.ShapeDtypeStruct((BB, 2), jnp.int32),
            jax.ShapeDtypeStruct((BB, 2), jnp.float32),
            jax.ShapeDtypeStruct((1, EE), jnp.int32),
        ],
        scratch_shapes=[pltpu.VMEM((1, EE), jnp.int32)],
        compiler_params=pltpu.CompilerParams(
            dimension_semantics=("arbitrary",)),
    )(x, Wg_p, bg.reshape(1, EE))


@functools.partial(
    pl.kernel, mesh=_sc_mesh,
    out_type=[
        jax.ShapeDtypeStruct((SLOTS,), jnp.int32),
        jax.ShapeDtypeStruct((NA + 128,), jnp.int32),
        jax.ShapeDtypeStruct((144,), jnp.int32),
    ],
    scratch_types=[
        pltpu.VMEM((NA,), jnp.int32),
        pltpu.VMEM((16,), jnp.int32),
        pltpu.VMEM((CAP,), jnp.int32),
        pltpu.VMEM((34, 128), jnp.int32),
        pltpu.VMEM((16,), jnp.int32),
        pltpu.VMEM_SHARED((NA + 128,), jnp.int32),
        pltpu.SemaphoreType.DMA,
    ],
    compiler_params=_sc_params,
)
def _k2a(eidx_hbm, hist_hbm, tok_hbm, pos_hbm, tile_hbm,
        a_v, hist_v, pv_v, buf2d_v, tv_v, shpos, sem):
    cid = lax.axis_index("c")
    sid = lax.axis_index("s")
    w = cid * 16 + sid

    @pl.when(w < 16)
    def _():
        lane = lax.iota(jnp.int32, 16)
        pltpu.sync_copy(eidx_hbm, a_v)
        pltpu.sync_copy(hist_hbm, hist_v)
        hist = hist_v[...]
        pc = jnp.bitwise_and(hist + 127, -128)
        incl = plsc.cumsum(pc)
        off_all = incl - pc
        off_s = jnp.sum(jnp.where(lane == w, off_all, 0))
        ntiles_s = jnp.sum(jnp.where(lane == w, pc, 0)) >> 7

        def comp_body(g, cur):
            av = a_v[pl.ds(g * 16, 16)]
            m = av == w
            r = plsc.cumsum(m.astype(jnp.int32))
            p = cur + r - 1
            plsc.store_scatter(buf2d_v, [p >> 7, p & 127], g * 16 + lane,
                               mask=m)
            return cur + plsc.all_reduce_population_count(m)
        cur = plsc.parallel_loop(0, NA // 16, 1, unroll=8,
                                 carry=jnp.zeros((16,), jnp.int32))(comp_body)
        cnt_s = jnp.max(cur)

        def tail_body(j, _):
            p = cnt_s + j * 16 + lane
            plsc.store_scatter(buf2d_v, [p >> 7, p & 127], NA + j * 16 + lane)
            return 0
        lax.fori_loop(0, 9, tail_body, 0)

        @plsc.parallel_loop(0, ntiles_s * 8, 1, unroll=4)
        def pv_body(j):
            pv_v[pl.ds(j * 16, 16)] = off_s + j * 16 + lane

        def sc_body(j, _):
            pltpu.sync_copy(pv_v.at[pl.ds(j * 128, 128)],
                            shpos.at[buf2d_v.at[j]])
            return 0
        lax.fori_loop(0, ntiles_s, sc_body, 0)

        def tk_body(j, _):
            dst = pl.multiple_of(off_s + j * 128, 128)
            pltpu.sync_copy(buf2d_v.at[j], tok_hbm.at[pl.ds(dst, 128)])
            return 0
        lax.fori_loop(0, ntiles_s, tk_body, 0)

        tv_v[...] = jnp.broadcast_to(w, (16,)).astype(jnp.int32)
        for j2 in range(2):
            tidx = jnp.where(j2 * 16 + lane < ntiles_s,
                             (off_s >> 7) + j2 * 16 + lane, 128 + lane)
            pltpu.async_copy(tv_v, tile_hbm.at[tidx], sem).wait()

    plsc.subcore_barrier()

    @pl.when(w == 0)
    def _():
        pltpu.sync_copy(shpos, pos_hbm)


@functools.partial(
    pl.kernel, mesh=_sc_mesh,
    out_type=jax.ShapeDtypeStruct((SLOTS, DPAD), jnp.float32),
    scratch_types=[
        pltpu.VMEM((80,), jnp.int32),
        pltpu.VMEM((80, DPAD), jnp.float32),
        pltpu.SemaphoreType.DMA,
    ],
    compiler_params=_sc_params,
)
def _k2b(tok_hbm, x_hbm, xs_hbm, idx_v, rows_v, sem):
    w = lax.axis_index("s") * 2 + lax.axis_index("c")
    base = w * (SLOTS // 32)
    for cch in range(4):
        pltpu.sync_copy(tok_hbm.at[pl.ds(base + cch * 80, 80)], idx_v)
        for j in range(5):
            idx_v[pl.ds(j * 16, 16)] = (idx_v[pl.ds(j * 16, 16)] >> 1) & (BB - 1)
        pltpu.async_copy(x_hbm.at[idx_v], rows_v, sem).wait()
        pltpu.sync_copy(rows_v, xs_hbm.at[pl.ds(base + cch * 80, 80)])


def _k3_body(tile_ref, xs_ref, W1_ref, b1_ref, W2_ref, b2_ref, Wout_ref,
             bout_ref, z_ref):
    t = pl.program_id(0)
    e = tile_ref[t] & (EE - 1)
    x = xs_ref[...]
    h = jnp.dot(x, W1_ref[e], preferred_element_type=jnp.float32) + b1_ref[e]
    h = jnp.maximum(h, 0.0)
    h = jnp.dot(h, W2_ref[e], preferred_element_type=jnp.float32) + b2_ref[e]
    h = jnp.maximum(h, 0.0)
    z_ref[...] = (jnp.dot(h, Wout_ref[...], preferred_element_type=jnp.float32)
                  + bout_ref[...])


def _k3(tile_e, xs, W1_p, b1, W2, b2, Wout, bout):
    return pl.pallas_call(
        _k3_body,
        grid=(NTILES,),
        in_specs=[
            pl.BlockSpec(memory_space=pltpu.MemorySpace.SMEM),
            pl.BlockSpec((TM, DPAD), lambda t: (t, 0)),
            pl.BlockSpec((EE, DPAD, DBLK), lambda t: (0, 0, 0)),
            pl.BlockSpec((EE, 1, DBLK), lambda t: (0, 0, 0)),
            pl.BlockSpec((EE, DBLK, DBLK), lambda t: (0, 0, 0)),
            pl.BlockSpec((EE, 1, DBLK), lambda t: (0, 0, 0)),
            pl.BlockSpec((DBLK, 1), lambda t: (0, 0)),
            pl.BlockSpec((1, 1), lambda t: (0, 0)),
        ],
        out_specs=pl.BlockSpec((TM, 1), lambda t: (t, 0)),
        out_shape=jax.ShapeDtypeStruct((SLOTS, 1), jnp.float32),
        compiler_params=pltpu.CompilerParams(
            dimension_semantics=("arbitrary",)),
    )(tile_e, xs, W1_p, b1.reshape(EE, 1, DBLK), W2,
      b2.reshape(EE, 1, DBLK), Wout, bout.reshape(1, 1))


@functools.partial(
    pl.kernel, mesh=_sc_mesh,
    out_type=jax.ShapeDtypeStruct((BB,), jnp.float32),
    scratch_types=[
        pltpu.VMEM((SLOTS,), jnp.float32),
        pltpu.VMEM((256,), jnp.int32),
        pltpu.VMEM((256,), jnp.float32),
        pltpu.VMEM((256,), jnp.float32),
        pltpu.VMEM((128,), jnp.float32),
        pltpu.SemaphoreType.DMA,
    ],
    compiler_params=_sc_params,
)
def _k4(z_hbm, pos_hbm, g_hbm, out_hbm, z_v, pos_v, g_v, val_v, out_v, sem):
    w = lax.axis_index("s") * 2 + lax.axis_index("c")
    lane = lax.iota(jnp.int32, 16)
    pltpu.sync_copy(z_hbm, z_v)
    pltpu.sync_copy(pos_hbm.at[pl.ds(w * 256, 256)], pos_v)
    pltpu.sync_copy(g_hbm.at[pl.ds(w * 256, 256)], g_v)
    for j in range(16):
        idx = pos_v[pl.ds(j * 16, 16)]
        val_v[pl.ds(j * 16, 16)] = (
            plsc.load_gather(z_v, [idx]) * g_v[pl.ds(j * 16, 16)])
    for j in range(8):
        idxe = j * 32 + 2 * lane
        oe = plsc.load_gather(val_v, [idxe])
        oo = plsc.load_gather(val_v, [idxe + 1])
        out_v[pl.ds(j * 16, 16)] = oe + oo
    pltpu.sync_copy(out_v, out_hbm.at[pl.ds(w * 128, 128)])


@jax.jit
def _moe_sparse(x_num, x_cat, Wg, bg, W1, b1, W2, b2, Wout, bout):
    B = x_num.shape[0]
    oh = jax.nn.one_hot(x_cat, 16, dtype=jnp.float32)
    oh = oh.reshape(B, x_cat.shape[1] * 16)
    x = jnp.concatenate(
        [x_num, oh, jnp.zeros((B, DPAD - 544), jnp.float32)], axis=1)
    Wg_p = jnp.concatenate(
        [Wg, jnp.zeros((DPAD - 544, EE), jnp.float32)], axis=0)
    W1_p = jnp.concatenate(
        [W1, jnp.zeros((EE, DPAD - 544, DBLK), jnp.float32)], axis=1)

    eidx, gates, hist = _k1(x, Wg_p, bg)
    tok, pos, tile_e = _k2a(eidx.reshape(-1), hist.reshape(-1))
    xs = _k2b(tok, x)
    z = _k3(tile_e, xs, W1_p, b1, W2, b2, Wout, bout)
    out = _k4(z.reshape(-1), pos, gates.reshape(-1))
    return out.reshape(B, 1, 1)


def kernel(x_num, x_cat, Wg, bg, W1, b1, W2, b2, Wout, bout):
    return _moe_sparse(x_num, x_cat, Wg, bg, W1, b1, W2, b2, Wout, bout)


# speedup vs baseline: 41.2508x; 1.0566x over previous
import functools

import jax
import jax.numpy as jnp
from jax import lax
from jax.experimental import pallas as pl
from jax.experimental.pallas import tpu as pltpu
from jax.experimental.pallas import tpu_sc as plsc

BB = 4096
EE = 16
NA = 8192
TM = 128
SLOTS = 10240
NTILES = SLOTS // TM
DPAD = 640
DBLK = 256
TB = 256
T1 = BB // TB
NW = 32
CH = NA // NW

_sc_mesh = plsc.VectorSubcoreMesh(core_axis_name="c", subcore_axis_name="s")
_sc_params = pltpu.CompilerParams(needs_layout_passes=False)


def _k1_body(x_ref, Wg_ref, bg_ref, eidx_ref, gates_ref, hists_ref):
    x = x_ref[...]
    logits = jnp.dot(x, Wg_ref[...], preferred_element_type=jnp.float32)
    logits = logits + bg_ref[...]
    m = jnp.max(logits, axis=1, keepdims=True)
    p = jnp.exp(logits - m)
    p = p / jnp.sum(p, axis=1, keepdims=True)
    lane = lax.broadcasted_iota(jnp.int32, (TB, EE), 1)
    m1 = jnp.max(p, axis=1, keepdims=True)
    a1 = jnp.min(jnp.where(p == m1, lane, EE), axis=1, keepdims=True)
    p2 = jnp.where(lane == a1, -1.0, p)
    m2 = jnp.max(p2, axis=1, keepdims=True)
    a2 = jnp.min(jnp.where(p2 == m2, lane, EE), axis=1, keepdims=True)
    s = m1 + m2
    eidx_ref[...] = jnp.concatenate([a1, a2], axis=1)
    gates_ref[...] = jnp.concatenate([m1 / s, m2 / s], axis=1)
    oh = (lane == a1).astype(jnp.int32) + (lane == a2).astype(jnp.int32)
    ch1 = jnp.sum(oh[:TB // 2], axis=0, keepdims=True)
    ch2 = jnp.sum(oh[TB // 2:], axis=0, keepdims=True)
    hists_ref[...] = jnp.concatenate([ch1, ch2], axis=0)[None]


def _k1(x, Wg_p, bg):
    return pl.pallas_call(
        _k1_body,
        grid=(T1,),
        in_specs=[
            pl.BlockSpec((TB, DPAD), lambda t: (t, 0)),
            pl.BlockSpec((DPAD, EE), lambda t: (0, 0)),
            pl.BlockSpec((1, EE), lambda t: (0, 0)),
        ],
        out_specs=[
            pl.BlockSpec((TB, 2), lambda t: (t, 0)),
            pl.BlockSpec((TB, 2), lambda t: (t, 0)),
            pl.BlockSpec((1, 2, EE), lambda t: (t, 0, 0)),
        ],
        out_shape=[
            jax.ShapeDtypeStruct((BB, 2), jnp.int32),
            jax.ShapeDtypeStruct((BB, 2), jnp.float32),
            jax.ShapeDtypeStruct((T1, 2, EE), jnp.int32),
        ],
        compiler_params=pltpu.CompilerParams(
            dimension_semantics=("arbitrary",)),
    )(x, Wg_p, bg.reshape(1, EE))


@functools.partial(
    pl.kernel, mesh=_sc_mesh,
    out_type=[
        jax.ShapeDtypeStruct((NA,), jnp.int32),
        jax.ShapeDtypeStruct((144,), jnp.int32),
    ],
    scratch_types=[
        pltpu.VMEM((CH,), jnp.int32),
        pltpu.VMEM((NW, EE), jnp.int32),
        pltpu.VMEM((EE,), jnp.int32),
        pltpu.VMEM((CH,), jnp.int32),
        pltpu.VMEM((16,), jnp.int32),
        pltpu.SemaphoreType.DMA,
    ],
    compiler_params=_sc_params,
)
def _k2a(eidx_hbm, hists_hbm, pos_hbm, tile_hbm,
         a_v, h32_v, bo_v, pv_v, tv_v, sem):
    w = lax.axis_index("c") * 16 + lax.axis_index("s")
    lane = lax.iota(jnp.int32, 16)
    pltpu.sync_copy(eidx_hbm.at[pl.ds(w * CH, CH)], a_v)
    pltpu.sync_copy(hists_hbm, h32_v)
    total = jnp.zeros((EE,), jnp.int32)
    cur0 = jnp.zeros((EE,), jnp.int32)
    for j in range(NW):
        row = h32_v[j]
        total = total + row
        cur0 = cur0 + jnp.where(j < w, row, 0)
    pc = jnp.bitwise_and(total + (TM - 1), -TM)
    incl = plsc.cumsum(pc)
    off = incl - pc
    bo_v[...] = off + cur0
    ones = jnp.ones((16,), jnp.int32)
    for g in range(CH // 16):
        av = a_v[pl.ds(g * 16, 16)]
        r, _ = plsc.scan_count(av)
        base = plsc.load_gather(bo_v, [av])
        pv_v[pl.ds(g * 16, 16)] = base + r - 1
        plsc.addupdate_scatter(bo_v, [av], ones)
    pltpu.sync_copy(pv_v, pos_hbm.at[pl.ds(w * CH, CH)])

    @pl.when(w < EE)
    def _():
        off_s = jnp.sum(jnp.where(lane == w, off, 0))
        ntiles_s = jnp.sum(jnp.where(lane == w, pc, 0)) >> 7
        tv_v[...] = jnp.broadcast_to(w, (16,)).astype(jnp.int32)
        for j2 in range(2):
            tidx = jnp.where(j2 * 16 + lane < ntiles_s,
                             (off_s >> 7) + j2 * 16 + lane, 128 + lane)
            pltpu.async_copy(tv_v, tile_hbm.at[tidx], sem).wait()


@functools.partial(
    pl.kernel, mesh=_sc_mesh,
    out_type=jax.ShapeDtypeStruct((SLOTS, DPAD), jnp.float32),
    scratch_types=[
        pltpu.VMEM((TM, DPAD), jnp.float32),
        pltpu.VMEM((CH,), jnp.int32),
        pltpu.VMEM((TM,), jnp.int32),
        pltpu.VMEM((TM,), jnp.int32),
        pltpu.SemaphoreType.DMA,
        pltpu.SemaphoreType.DMA,
    ],
    compiler_params=_sc_params,
)
def _k2b(x_hbm, pos_hbm, xs_hbm, rows_v, pos_v, ie_v, io_v, sem1, sem2):
    w = lax.axis_index("c") * 16 + lax.axis_index("s")
    lane = lax.iota(jnp.int32, 16)
    pltpu.sync_copy(x_hbm.at[pl.ds(w * TM, TM)], rows_v)
    pltpu.sync_copy(pos_hbm.at[pl.ds(w * CH, CH)], pos_v)
    for j in range(8):
        ie_v[pl.ds(j * 16, 16)] = plsc.load_gather(pos_v, [j * 32 + 2 * lane])
        io_v[pl.ds(j * 16, 16)] = plsc.load_gather(
            pos_v, [j * 32 + 2 * lane + 1])
    c1 = pltpu.async_copy(rows_v, xs_hbm.at[ie_v], sem1)
    c2 = pltpu.async_copy(rows_v, xs_hbm.at[io_v], sem2)
    c1.wait()
    c2.wait()


def _k3_body(tile_ref, xs_ref, W1_ref, b1_ref, W2_ref, b2_ref, Wout_ref,
             bout_ref, z_ref):
    t = pl.program_id(0)
    e = tile_ref[t] & (EE - 1)
    x = xs_ref[...]
    h = jnp.dot(x, W1_ref[e], preferred_element_type=jnp.float32) + b1_ref[e]
    h = jnp.maximum(h, 0.0)
    h = jnp.dot(h, W2_ref[e], preferred_element_type=jnp.float32) + b2_ref[e]
    h = jnp.maximum(h, 0.0)
    z_ref[...] = (jnp.dot(h, Wout_ref[...], preferred_element_type=jnp.float32)
                  + bout_ref[...])


def _k3(tile_e, xs, W1_p, b1, W2, b2, Wout, bout):
    return pl.pallas_call(
        _k3_body,
        grid=(NTILES,),
        in_specs=[
            pl.BlockSpec(memory_space=pltpu.MemorySpace.SMEM),
            pl.BlockSpec((TM, DPAD), lambda t: (t, 0)),
            pl.BlockSpec((EE, DPAD, DBLK), lambda t: (0, 0, 0)),
            pl.BlockSpec((EE, 1, DBLK), lambda t: (0, 0, 0)),
            pl.BlockSpec((EE, DBLK, DBLK), lambda t: (0, 0, 0)),
            pl.BlockSpec((EE, 1, DBLK), lambda t: (0, 0, 0)),
            pl.BlockSpec((DBLK, 1), lambda t: (0, 0)),
            pl.BlockSpec((1, 1), lambda t: (0, 0)),
        ],
        out_specs=pl.BlockSpec((TM, 1), lambda t: (t, 0)),
        out_shape=jax.ShapeDtypeStruct((SLOTS, 1), jnp.float32),
        compiler_params=pltpu.CompilerParams(
            dimension_semantics=("arbitrary",)),
    )(tile_e, xs, W1_p, b1.reshape(EE, 1, DBLK), W2,
      b2.reshape(EE, 1, DBLK), Wout, bout.reshape(1, 1))


@functools.partial(
    pl.kernel, mesh=_sc_mesh,
    out_type=jax.ShapeDtypeStruct((BB,), jnp.float32),
    scratch_types=[
        pltpu.VMEM((SLOTS,), jnp.float32),
        pltpu.VMEM((CH,), jnp.int32),
        pltpu.VMEM((CH,), jnp.float32),
        pltpu.VMEM((CH,), jnp.float32),
        pltpu.VMEM((TM,), jnp.float32),
        pltpu.SemaphoreType.DMA,
    ],
    compiler_params=_sc_params,
)
def _k4(z_hbm, pos_hbm, g_hbm, out_hbm, z_v, pos_v, g_v, val_v, out_v, sem):
    w = lax.axis_index("c") * 16 + lax.axis_index("s")
    lane = lax.iota(jnp.int32, 16)
    pltpu.sync_copy(z_hbm, z_v)
    pltpu.sync_copy(pos_hbm.at[pl.ds(w * CH, CH)], pos_v)
    pltpu.sync_copy(g_hbm.at[pl.ds(w * CH, CH)], g_v)
    for j in range(CH // 16):
        idx = pos_v[pl.ds(j * 16, 16)]
        val_v[pl.ds(j * 16, 16)] = (
            plsc.load_gather(z_v, [idx]) * g_v[pl.ds(j * 16, 16)])
    for j in range(TM // 16):
        idxe = j * 32 + 2 * lane
        oe = plsc.load_gather(val_v, [idxe])
        oo = plsc.load_gather(val_v, [idxe + 1])
        out_v[pl.ds(j * 16, 16)] = oe + oo
    pltpu.sync_copy(out_v, out_hbm.at[pl.ds(w * TM, TM)])


@jax.jit
def _moe_sparse(x_num, x_cat, Wg, bg, W1, b1, W2, b2, Wout, bout):
    B = x_num.shape[0]
    oh = jax.nn.one_hot(x_cat, 16, dtype=jnp.float32)
    oh = oh.reshape(B, x_cat.shape[1] * 16)
    x = jnp.concatenate(
        [x_num, oh, jnp.zeros((B, DPAD - 544), jnp.float32)], axis=1)
    Wg_p = jnp.concatenate(
        [Wg, jnp.zeros((DPAD - 544, EE), jnp.float32)], axis=0)
    W1_p = jnp.concatenate(
        [W1, jnp.zeros((EE, DPAD - 544, DBLK), jnp.float32)], axis=1)

    eidx, gates, hists = _k1(x, Wg_p, bg)
    pos, tile_e = _k2a(eidx.reshape(-1), hists.reshape(NW, EE))
    xs = _k2b(x, pos)
    z = _k3(tile_e, xs, W1_p, b1, W2, b2, Wout, bout)
    out = _k4(z.reshape(-1), pos, gates.reshape(-1))
    return out.reshape(B, 1, 1)


def kernel(x_num, x_cat, Wg, bg, W1, b1, W2, b2, Wout, bout):
    return _moe_sparse(x_num, x_cat, Wg, bg, W1, b1, W2, b2, Wout, bout)
